# baseline (speedup 1.0000x reference)
import functools

import jax
import jax.numpy as jnp
from jax import lax
from jax.experimental import pallas as pl
from jax.experimental.pallas import tpu as pltpu
from jax.experimental.pallas import tpu_sc as plsc

EMB_DIM = 1024
SEQ_LEN = 8192

_NC = 2
_NS = 16
_NW = _NC * _NS
_CHUNK = 16
_NBUF = 6

_ROWS_PER_CORE = SEQ_LEN // _NC
_R_TILE = 128
_SP_ROWS = _ROWS_PER_CORE - (_NS - 1) * _R_TILE
_SP_N = 16
_SP_CHUNK = _SP_ROWS // _SP_N
_IDX_ROWS = SEQ_LEN // _CHUNK + 16
_B_PER_W = SEQ_LEN // _NW
_N_CHUNKS = _B_PER_W // _CHUNK


def _gather_ring(table_hbm, out_hbm, idx_v, bufs, gsems, ssems,
                 row_base, n_chunks):
    def start_gather(g):
        return pltpu.async_copy(
            table_hbm.at[idx_v.at[g]], bufs[g % _NBUF], gsems[g % _NBUF])

    def start_scatter(j):
        return pltpu.async_copy(
            bufs[j % _NBUF],
            out_hbm.at[pl.ds(row_base + j * _CHUNK, _CHUNK)],
            ssems[j % _NBUF])

    row_base = pl.multiple_of(row_base, 128)
    gh, sh = {}, {}
    for g in range(min(_NBUF - 1, n_chunks)):
        gh[g] = start_gather(g)
    for j in range(n_chunks):
        gh[j].wait()
        sh[j] = start_scatter(j)
        g = j + _NBUF - 1
        if g < n_chunks:
            if g - _NBUF >= 0:
                sh[g - _NBUF].wait()
            gh[g] = start_gather(g)
    for j in range(max(0, n_chunks - _NBUF), n_chunks):
        sh[j].wait()


def _sc_lookup(table, idx2, tail):
    mesh = plsc.VectorSubcoreMesh(core_axis_name="c", subcore_axis_name="s")

    @functools.partial(
        pl.kernel,
        mesh=mesh,
        out_type=jax.ShapeDtypeStruct((SEQ_LEN, EMB_DIM), jnp.float32),
        scratch_types=[
            pltpu.VMEM((_N_CHUNKS, _CHUNK), jnp.int32),
            pltpu.VMEM((16,), jnp.int32),
            *[pltpu.VMEM((_CHUNK, EMB_DIM), jnp.float32) for _ in range(_NBUF)],
            pltpu.VMEM_SHARED((2, _SP_CHUNK, EMB_DIM), jnp.float32),
            *[pltpu.SemaphoreType.DMA for _ in range(2 * _NBUF + 4)],
        ],
    )
    def k(table_hbm, idx2_hbm, tail_hbm, out_hbm, idx_v, tail_v, *scratch):
        bufs = scratch[:_NBUF]
        spbuf = scratch[_NBUF]
        gsems = scratch[_NBUF + 1:2 * _NBUF + 1]
        ssems = scratch[2 * _NBUF + 1:3 * _NBUF + 1]
        spsems = scratch[3 * _NBUF + 1:]
        cid = lax.axis_index("c")
        sid = lax.axis_index("s")

        pltpu.sync_copy(tail_hbm, tail_v)
        last = tail_v[...][15]
        is_id = last == SEQ_LEN - 1

        @pl.when(jnp.logical_and(is_id, sid == 0))
        def _():
            cb = pl.multiple_of(cid * _ROWS_PER_CORE, 1024)
            gh, sh = {}, {}

            def sp_gather(g):
                return pltpu.async_copy(
                    table_hbm.at[pl.ds(cb + g * _SP_CHUNK, _SP_CHUNK)],
                    spbuf.at[g % 2], spsems[g % 2])

            def sp_scatter(j):
                return pltpu.async_copy(
                    spbuf.at[j % 2],
                    out_hbm.at[pl.ds(cb + j * _SP_CHUNK, _SP_CHUNK)],
                    spsems[2 + j % 2])

            gh[0] = sp_gather(0)
            for j in range(_SP_N):
                gh[j].wait()
                sh[j] = sp_scatter(j)
                g = j + 1
                if g < _SP_N:
                    if g - 2 >= 0:
                        sh[g - 2].wait()
                    gh[g] = sp_gather(g)
            for j in range(max(0, _SP_N - 2), _SP_N):
                sh[j].wait()

        @pl.when(jnp.logical_and(is_id, sid > 0))
        def _():
            tb = cid * _ROWS_PER_CORE + _SP_ROWS + (sid - 1) * _R_TILE
            n = _R_TILE // _CHUNK
            off = pl.multiple_of(tb // _CHUNK, 8)
            pltpu.sync_copy(idx2_hbm.at[pl.ds(off, 16)], idx_v)
            _gather_ring(table_hbm, out_hbm, idx_v, bufs, gsems, ssems, tb, n)

        @pl.when(jnp.logical_not(is_id))
        def _():
            wid = sid * _NC + cid
            wb = wid * _B_PER_W
            off = pl.multiple_of(wb // _CHUNK, 16)
            pltpu.sync_copy(idx2_hbm.at[pl.ds(off, _N_CHUNKS)], idx_v)
            _gather_ring(table_hbm, out_hbm, idx_v, bufs, gsems, ssems,
                         wb, _N_CHUNKS)

    return k(table, idx2, tail)


def kernel(seq_len, pos_emb):
    positions = jnp.arange(0, SEQ_LEN, dtype=jnp.int32)
    positions = jnp.minimum(positions, jnp.asarray(seq_len, dtype=jnp.int32) - 1)
    tail = positions[SEQ_LEN - 16:]
    positions = jnp.concatenate(
        [positions, jnp.zeros(16 * _CHUNK, jnp.int32)])
    idx2 = positions.reshape(_IDX_ROWS, _CHUNK)
    return _sc_lookup(pos_emb, idx2, tail)


# speedup vs baseline: 1.1702x; 1.1702x over previous
import functools

import jax
import jax.numpy as jnp
from jax import lax
from jax.experimental import pallas as pl
from jax.experimental.pallas import tpu as pltpu
from jax.experimental.pallas import tpu_sc as plsc

EMB_DIM = 1024
SEQ_LEN = 8192

_NC = 2
_NS = 16
_NW = _NC * _NS
_CHUNK = 16
_NBUF = 4

_ROWS_PER_CORE = SEQ_LEN // _NC
_R_TILE = 160
_SP_ROWS = _ROWS_PER_CORE - (_NS - 1) * _R_TILE
_SP_N = 4
_SP_CHUNK = _SP_ROWS // _SP_N
_IDX_ROWS = SEQ_LEN // _CHUNK + 24
_B_PER_W = SEQ_LEN // _NW
_N_CHUNKS = _B_PER_W // _CHUNK


def _gather_ring(table_hbm, out_hbm, idx_v, idx_skew, bufs, gsems, ssems,
                 row_base, n_chunks):
    def start_gather(g):
        return pltpu.async_copy(
            table_hbm.at[idx_v.at[idx_skew + g]],
            bufs[g % _NBUF], gsems[g % _NBUF])

    def start_scatter(j):
        return pltpu.async_copy(
            bufs[j % _NBUF],
            out_hbm.at[pl.ds(row_base + j * _CHUNK, _CHUNK)],
            ssems[j % _NBUF])

    gh, sh = {}, {}
    for g in range(min(_NBUF - 1, n_chunks)):
        gh[g] = start_gather(g)
    for j in range(n_chunks):
        gh[j].wait()
        sh[j] = start_scatter(j)
        g = j + _NBUF - 1
        if g < n_chunks:
            if g - _NBUF >= 0:
                sh[g - _NBUF].wait()
            gh[g] = start_gather(g)
    for j in range(max(0, n_chunks - _NBUF), n_chunks):
        sh[j].wait()


def _sc_lookup(table, idx2, tail):
    mesh = plsc.VectorSubcoreMesh(core_axis_name="c", subcore_axis_name="s")

    @functools.partial(
        pl.kernel,
        mesh=mesh,
        out_type=jax.ShapeDtypeStruct((SEQ_LEN, EMB_DIM), jnp.float32),
        scratch_types=[
            pltpu.VMEM((24, _CHUNK), jnp.int32),
            pltpu.VMEM((16,), jnp.int32),
            *[pltpu.VMEM((_CHUNK, EMB_DIM), jnp.float32) for _ in range(_NBUF)],
            pltpu.VMEM_SHARED((2, _SP_CHUNK, EMB_DIM), jnp.float32),
            *[pltpu.SemaphoreType.DMA for _ in range(2 * _NBUF + 4)],
        ],
    )
    def k(table_hbm, idx2_hbm, tail_hbm, out_hbm, idx_v, tail_v, *scratch):
        bufs = scratch[:_NBUF]
        spbuf = scratch[_NBUF]
        gsems = scratch[_NBUF + 1:2 * _NBUF + 1]
        ssems = scratch[2 * _NBUF + 1:3 * _NBUF + 1]
        spsems = scratch[3 * _NBUF + 1:]
        cid = lax.axis_index("c")
        sid = lax.axis_index("s")

        pltpu.sync_copy(tail_hbm, tail_v)
        last = tail_v[...][15]
        is_id = last == SEQ_LEN - 1

        @pl.when(jnp.logical_and(is_id, sid == 0))
        def _():
            cb = pl.multiple_of(cid * _ROWS_PER_CORE, 1024)
            gh, sh = {}, {}

            def sp_gather(g):
                return pltpu.async_copy(
                    table_hbm.at[pl.ds(cb + g * _SP_CHUNK, _SP_CHUNK)],
                    spbuf.at[g % 2], spsems[g % 2])

            def sp_scatter(j):
                return pltpu.async_copy(
                    spbuf.at[j % 2],
                    out_hbm.at[pl.ds(cb + j * _SP_CHUNK, _SP_CHUNK)],
                    spsems[2 + j % 2])

            gh[0] = sp_gather(0)
            for j in range(_SP_N):
                gh[j].wait()
                sh[j] = sp_scatter(j)
                g = j + 1
                if g < _SP_N:
                    if g - 2 >= 0:
                        sh[g - 2].wait()
                    gh[g] = sp_gather(g)
            for j in range(max(0, _SP_N - 2), _SP_N):
                sh[j].wait()

        @pl.when(jnp.logical_and(is_id, sid > 0))
        def _():
            tb = cid * _ROWS_PER_CORE + _SP_ROWS + (sid - 1) * _R_TILE
            n = _R_TILE // _CHUNK
            row0 = tb // _CHUNK
            off_al = pl.multiple_of((row0 // 8) * 8, 8)
            skew = row0 - off_al
            pltpu.sync_copy(idx2_hbm.at[pl.ds(off_al, 24)], idx_v)
            _gather_ring(table_hbm, out_hbm, idx_v, skew, bufs, gsems,
                         ssems, tb, n)

        @pl.when(jnp.logical_not(is_id))
        def _():
            wid = sid * _NC + cid
            wb = wid * _B_PER_W
            off = pl.multiple_of(wb // _CHUNK, 16)
            pltpu.sync_copy(
                idx2_hbm.at[pl.ds(off, _N_CHUNKS)],
                idx_v.at[pl.ds(0, _N_CHUNKS)])
            _gather_ring(table_hbm, out_hbm, idx_v, 0, bufs, gsems, ssems,
                         wb, _N_CHUNKS)

    return k(table, idx2, tail)


def kernel(seq_len, pos_emb):
    positions = jnp.arange(0, SEQ_LEN, dtype=jnp.int32)
    positions = jnp.minimum(positions, jnp.asarray(seq_len, dtype=jnp.int32) - 1)
    tail = positions[SEQ_LEN - 16:]
    positions = jnp.concatenate(
        [positions, jnp.zeros(24 * _CHUNK, jnp.int32)])
    idx2 = positions.reshape(_IDX_ROWS, _CHUNK)
    return _sc_lookup(pos_emb, idx2, tail)


# speedup vs baseline: 1.2069x; 1.0314x over previous
import functools

import jax
import jax.numpy as jnp
from jax import lax
from jax.experimental import pallas as pl
from jax.experimental.pallas import tpu as pltpu
from jax.experimental.pallas import tpu_sc as plsc

EMB_DIM = 1024
SEQ_LEN = 8192

_NC = 2
_NS = 16
_NW = _NC * _NS
_B_PER_W = SEQ_LEN // _NW
_CHUNK = 16
_N_CHUNKS = _B_PER_W // _CHUNK
_NBUF = 7


def _sc_gather(table, idx):
    mesh = plsc.VectorSubcoreMesh(core_axis_name="c", subcore_axis_name="s")

    @functools.partial(
        pl.kernel,
        mesh=mesh,
        out_type=jax.ShapeDtypeStruct((SEQ_LEN, EMB_DIM), jnp.float32),
        scratch_types=[
            pltpu.VMEM((_N_CHUNKS, _CHUNK), jnp.int32),
            *[pltpu.VMEM((_CHUNK, EMB_DIM), jnp.float32) for _ in range(_NBUF)],
            *[pltpu.SemaphoreType.DMA for _ in range(2 * _NBUF)],
        ],
    )
    def k(table_hbm, idx_hbm, out_hbm, idx_v, *scratch):
        bufs = scratch[:_NBUF]
        gsems = scratch[_NBUF:2 * _NBUF]
        ssems = scratch[2 * _NBUF:]
        wid = lax.axis_index("s") * _NC + lax.axis_index("c")
        base = wid * _B_PER_W
        pltpu.sync_copy(idx_hbm.at[wid], idx_v)

        def start_gather(g):
            return pltpu.async_copy(
                table_hbm.at[idx_v.at[g]], bufs[g % _NBUF], gsems[g % _NBUF])

        def start_scatter(j):
            return pltpu.async_copy(
                bufs[j % _NBUF],
                out_hbm.at[pl.ds(base + j * _CHUNK, _CHUNK)],
                ssems[j % _NBUF])

        gh, sh = {}, {}
        for g in range(min(_NBUF - 1, _N_CHUNKS)):
            gh[g] = start_gather(g)
        for j in range(_N_CHUNKS):
            gh[j].wait()
            sh[j] = start_scatter(j)
            g = j + _NBUF - 1
            if g < _N_CHUNKS:
                if g - _NBUF >= 0:
                    sh[g - _NBUF].wait()
                gh[g] = start_gather(g)
        for j in range(max(0, _N_CHUNKS - _NBUF), _N_CHUNKS):
            sh[j].wait()

    return k(table, idx)


def kernel(seq_len, pos_emb):
    positions = jnp.arange(0, SEQ_LEN, dtype=jnp.int32)
    positions = jnp.minimum(positions, jnp.asarray(seq_len, dtype=jnp.int32) - 1)
    idx = positions.reshape(_NW, _N_CHUNKS, _CHUNK)
    return _sc_gather(pos_emb, idx)


# speedup vs baseline: 1.2114x; 1.0038x over previous
import functools

import jax
import jax.numpy as jnp
from jax import lax
from jax.experimental import pallas as pl
from jax.experimental.pallas import tpu as pltpu
from jax.experimental.pallas import tpu_sc as plsc

EMB_DIM = 1024
SEQ_LEN = 8192

_NC = 2
_NS = 16
_NW = _NC * _NS
_B_PER_W = SEQ_LEN // _NW
_CHUNK = 16
_N_CHUNKS = _B_PER_W // _CHUNK
_NBUF = 6


def _sc_gather(table, idx):
    mesh = plsc.VectorSubcoreMesh(core_axis_name="c", subcore_axis_name="s")

    @functools.partial(
        pl.kernel,
        mesh=mesh,
        out_type=jax.ShapeDtypeStruct((SEQ_LEN, EMB_DIM), jnp.float32),
        scratch_types=[
            pltpu.VMEM((_N_CHUNKS, _CHUNK), jnp.int32),
            *[pltpu.VMEM((_CHUNK, EMB_DIM), jnp.float32) for _ in range(_NBUF)],
            *[pltpu.SemaphoreType.DMA for _ in range(2 * _NBUF)],
        ],
    )
    def k(table_hbm, idx_hbm, out_hbm, idx_v, *scratch):
        bufs = scratch[:_NBUF]
        gsems = scratch[_NBUF:2 * _NBUF]
        ssems = scratch[2 * _NBUF:]
        wid = lax.axis_index("s") * _NC + lax.axis_index("c")
        base = wid * _B_PER_W
        pltpu.sync_copy(idx_hbm.at[wid], idx_v)

        def start_gather(g):
            return pltpu.async_copy(
                table_hbm.at[idx_v.at[g]], bufs[g % _NBUF], gsems[g % _NBUF])

        def start_scatter(j):
            return pltpu.async_copy(
                bufs[j % _NBUF],
                out_hbm.at[pl.ds(base + j * _CHUNK, _CHUNK)],
                ssems[j % _NBUF])

        gh, sh = {}, {}
        for g in range(min(_NBUF - 1, _N_CHUNKS)):
            gh[g] = start_gather(g)
        for j in range(_N_CHUNKS):
            gh[j].wait()
            sh[j] = start_scatter(j)
            g = j + _NBUF - 1
            if g < _N_CHUNKS:
                if g - _NBUF >= 0:
                    sh[g - _NBUF].wait()
                gh[g] = start_gather(g)
        for j in range(max(0, _N_CHUNKS - _NBUF), _N_CHUNKS):
            sh[j].wait()

    return k(table, idx)


def kernel(seq_len, pos_emb):
    positions = jnp.arange(0, SEQ_LEN, dtype=jnp.int32)
    positions = jnp.minimum(positions, jnp.asarray(seq_len, dtype=jnp.int32) - 1)
    idx = positions.reshape(_NW, _N_CHUNKS, _CHUNK)
    return _sc_gather(pos_emb, idx)
